# Initial kernel scaffold; baseline (speedup 1.0000x reference)
import jax
import jax.numpy as jnp
from jax.experimental import pallas as pl


def kernel(x, table):
    raise NotImplementedError("write your pallas kernel here")


# speedup vs baseline: 2.7327x; 2.7327x over previous
import jax
import jax.numpy as jnp
from jax import lax
from jax.experimental import pallas as pl
from jax.experimental.pallas import tpu as pltpu
from jax.experimental.pallas import tpu_sc as plsc

_NC = 2
_NS = 16
_NW = _NC * _NS


def _gather_kernel(b_total: int, d: int, chunk: int):
    b_per_w = b_total // _NW
    n_chunks = b_per_w // chunk
    mesh = plsc.VectorSubcoreMesh(core_axis_name="c", subcore_axis_name="s")

    def body(idx_hbm, table_hbm, out_hbm, idx_v, rows_v, sem):
        wid = lax.axis_index("s") * _NC + lax.axis_index("c")
        base = wid * b_per_w
        pltpu.sync_copy(idx_hbm.at[wid], idx_v)

        def step(j, carry):
            pltpu.async_copy(table_hbm.at[idx_v.at[j]], rows_v, sem).wait()
            pltpu.sync_copy(rows_v, out_hbm.at[pl.ds(base + j * chunk, chunk)])
            return carry

        lax.fori_loop(0, n_chunks, step, 0, unroll=False)

    return pl.kernel(
        body,
        out_type=jax.ShapeDtypeStruct((b_total, d), jnp.float32),
        mesh=mesh,
        scratch_types=[
            pltpu.VMEM((n_chunks, chunk), jnp.int32),
            pltpu.VMEM((chunk, d), jnp.float32),
            pltpu.SemaphoreType.DMA,
        ],
    )


def kernel(x, table):
    batch, hist = x.shape
    vocab, d = table.shape
    b_total = batch * hist
    chunk = 128
    idx3 = x.reshape(_NW, (b_total // _NW) // chunk, chunk)
    out = _gather_kernel(b_total, d, chunk)(idx3, table)
    return out.reshape(batch, hist, d)


# speedup vs baseline: 2.7894x; 1.0207x over previous
import jax
import jax.numpy as jnp
from jax import lax
from jax.experimental import pallas as pl
from jax.experimental.pallas import tpu as pltpu
from jax.experimental.pallas import tpu_sc as plsc

_NC = 2
_NS = 16
_NW = _NC * _NS


def _gather_kernel(b_total: int, d: int, chunk: int, nbuf: int):
    b_per_w = b_total // _NW
    n_chunks = b_per_w // chunk
    n_groups = n_chunks // nbuf
    mesh = plsc.VectorSubcoreMesh(core_axis_name="c", subcore_axis_name="s")

    def body(idx_hbm, table_hbm, out_hbm, idx_v, *scratch):
        bufs = scratch[:nbuf]
        gsems = scratch[nbuf:2 * nbuf]
        wsems = scratch[2 * nbuf:3 * nbuf]
        wid = lax.axis_index("s") * _NC + lax.axis_index("c")
        base = wid * b_per_w
        pltpu.sync_copy(idx_hbm.at[wid], idx_v)

        def gather(j, b):
            return pltpu.make_async_copy(
                table_hbm.at[idx_v.at[j]], bufs[b], gsems[b])

        def write(j, b):
            return pltpu.make_async_copy(
                bufs[b], out_hbm.at[pl.ds(base + j * chunk, chunk)], wsems[b])

        for b in range(nbuf):
            gather(b, b).start()

        def group(g, carry):
            for b in range(nbuf):
                j = g * nbuf + b
                gather(j, b).wait()
                write(j, b).start()
            for b in range(nbuf):
                j = g * nbuf + b
                write(j, b).wait()
                gather(j + nbuf, b).start()
            return carry

        lax.fori_loop(0, n_groups - 1, group, 0, unroll=False)

        g = n_groups - 1
        for b in range(nbuf):
            j = g * nbuf + b
            gather(j, b).wait()
            write(j, b).start()
        for b in range(nbuf):
            write(g * nbuf + b, b).wait()

    return pl.kernel(
        body,
        out_type=jax.ShapeDtypeStruct((b_total, d), jnp.float32),
        mesh=mesh,
        scratch_types=(
            [pltpu.VMEM((n_chunks, chunk), jnp.int32)]
            + [pltpu.VMEM((chunk, d), jnp.float32) for _ in range(nbuf)]
            + [pltpu.SemaphoreType.DMA for _ in range(2 * nbuf)]
        ),
    )


def kernel(x, table):
    batch, hist = x.shape
    vocab, d = table.shape
    b_total = batch * hist
    chunk = 128
    nbuf = 5
    idx3 = x.reshape(_NW, (b_total // _NW) // chunk, chunk)
    out = _gather_kernel(b_total, d, chunk, nbuf)(idx3, table)
    return out.reshape(batch, hist, d)
